# Initial kernel scaffold; baseline (speedup 1.0000x reference)
import jax
import jax.numpy as jnp
from jax.experimental import pallas as pl


def kernel(x, edge_index, Wl0, bl0, Wr0, Wl1, bl1, Wr1, Wl2, bl2, Wr2, Wl3, bl3, Wr3):
    raise NotImplementedError("write your pallas kernel here")


# speedup vs baseline: 6.1386x; 6.1386x over previous
import functools

import jax
import jax.numpy as jnp
from jax import lax
from jax.experimental import pallas as pl
from jax.experimental.pallas import tpu as pltpu
from jax.experimental.pallas import tpu_sc as plsc

N = 10000
E = 320000
NC, NS = 2, 16
NW = NC * NS
CHUNK = 128
EPW = 80
E_PAD = NW * EPW * CHUNK
N_PAD = 10112
RPT = N_PAD // NS
DUMMY_DST = N

ROWS_BLK = 1000
GRID = N // ROWS_BLK


def _make_sc_aggregate(dp):
    mesh = plsc.VectorSubcoreMesh(core_axis_name="c", subcore_axis_name="s")

    @functools.partial(
        pl.kernel,
        mesh=mesh,
        compiler_params=pltpu.CompilerParams(use_tc_tiling_on_sc=False),
        out_type=jax.ShapeDtypeStruct((NC, N_PAD, dp), jnp.float32),
        scratch_types=[
            pltpu.VMEM((EPW, CHUNK), jnp.int32),
            pltpu.VMEM((EPW, CHUNK), jnp.int32),
            pltpu.VMEM((CHUNK, dp), jnp.float32),
            pltpu.VMEM((CHUNK, dp), jnp.float32),
            pltpu.VMEM_SHARED((N_PAD, dp), jnp.float32),
            pltpu.SemaphoreType.DMA,
            pltpu.SemaphoreType.DMA,
        ],
    )
    def sc_aggregate(proj_hbm, src_hbm, dst_hbm, zeros_hbm, out_hbm,
                     src_v, dst_v, buf_a, buf_b, accum, sem_g, sem_s):
        c = lax.axis_index("c")
        s = lax.axis_index("s")
        wid = s * NC + c
        row0 = s * RPT

        pltpu.sync_copy(zeros_hbm.at[pl.ds(row0, RPT)],
                        accum.at[pl.ds(row0, RPT)])
        pltpu.sync_copy(src_hbm.at[wid], src_v)
        pltpu.sync_copy(dst_hbm.at[wid], dst_v)
        plsc.subcore_barrier()

        def gather(j, buf):
            return pltpu.async_copy(proj_hbm.at[src_v.at[j]], buf, sem_g)

        gather(0, buf_a).wait()

        def body(j, _):
            def do(j, cur, nxt):
                g = gather(j + 1, nxt)
                pltpu.sync_copy(cur, accum.at[dst_v.at[j]], add=True)
                g.wait()

            lax.cond(j % 2 == 0,
                     lambda: do(j, buf_a, buf_b),
                     lambda: do(j, buf_b, buf_a))
            return 0

        lax.fori_loop(0, EPW - 1, body, 0)
        last = EPW - 1
        lax.cond(last % 2 == 0,
                 lambda: pltpu.sync_copy(buf_a, accum.at[dst_v.at[last]],
                                         add=True),
                 lambda: pltpu.sync_copy(buf_b, accum.at[dst_v.at[last]],
                                         add=True))
        plsc.subcore_barrier()

        pltpu.sync_copy(accum.at[pl.ds(row0, RPT)],
                        out_hbm.at[c].at[pl.ds(row0, RPT)])

    return sc_aggregate


_SC_AGG = {dp: _make_sc_aggregate(dp) for dp in (96, 64, 32, 16)}


def _tc0_body(x_ref, w_ref, ones_ref, o_ref):
    o_ref[...] = (jnp.dot(x_ref[...], w_ref[...],
                          preferred_element_type=jnp.float32) + ones_ref[...])


def _tc0(x, wl0p, ones_row):
    return pl.pallas_call(
        _tc0_body,
        grid=(GRID,),
        in_specs=[
            pl.BlockSpec((ROWS_BLK, 128), lambda r: (r, 0)),
            pl.BlockSpec((128, 96), lambda r: (0, 0)),
            pl.BlockSpec((1, 96), lambda r: (0, 0)),
        ],
        out_specs=pl.BlockSpec((ROWS_BLK, 96), lambda r: (r, 0)),
        out_shape=jax.ShapeDtypeStruct((N, 96), jnp.float32),
    )(x, wl0p, ones_row)


def _tc1_body(acc_ref, x_ref, wr_ref, bl_ref, wl_ref, h_ref, p_ref, cnt_ref):
    s = acc_ref[0] + acc_ref[1]
    cnt = jnp.maximum(s[:, 85:86], 1.0)
    h = (s[:, :85] / cnt
         + jnp.dot(x_ref[...], wr_ref[...], preferred_element_type=jnp.float32)
         + bl_ref[...])
    h_ref[...] = h
    p_ref[...] = jnp.dot(h, wl_ref[...], preferred_element_type=jnp.float32)
    cnt_ref[...] = cnt


def _tc1(acc0, x, wr0t, bl0, wl1p):
    return pl.pallas_call(
        _tc1_body,
        grid=(GRID,),
        in_specs=[
            pl.BlockSpec((NC, ROWS_BLK, 96), lambda r: (0, r, 0)),
            pl.BlockSpec((ROWS_BLK, 128), lambda r: (r, 0)),
            pl.BlockSpec((128, 85), lambda r: (0, 0)),
            pl.BlockSpec((1, 85), lambda r: (0, 0)),
            pl.BlockSpec((85, 64), lambda r: (0, 0)),
        ],
        out_specs=(
            pl.BlockSpec((ROWS_BLK, 85), lambda r: (r, 0)),
            pl.BlockSpec((ROWS_BLK, 64), lambda r: (r, 0)),
            pl.BlockSpec((ROWS_BLK, 1), lambda r: (r, 0)),
        ),
        out_shape=(
            jax.ShapeDtypeStruct((N, 85), jnp.float32),
            jax.ShapeDtypeStruct((N, 64), jnp.float32),
            jax.ShapeDtypeStruct((N, 1), jnp.float32),
        ),
    )(acc0, x, wr0t, bl0, wl1p)


def _make_tc_mid_body(dout_prev):
    def body(acc_ref, cnt_ref, h_ref, wr_ref, bl_ref, wl_ref, ho_ref, p_ref):
        s = acc_ref[0] + acc_ref[1]
        h = (s[:, :dout_prev] / cnt_ref[...]
             + jnp.dot(h_ref[...], wr_ref[...],
                       preferred_element_type=jnp.float32)
             + bl_ref[...])
        ho_ref[...] = h
        p_ref[...] = jnp.dot(h, wl_ref[...], preferred_element_type=jnp.float32)
    return body


def _tc_mid(acc, cnt, h, wrt, bl, wlp, dp_prev, dout_prev, din, dout, dp_next):
    return pl.pallas_call(
        _make_tc_mid_body(dout_prev),
        grid=(GRID,),
        in_specs=[
            pl.BlockSpec((NC, ROWS_BLK, dp_prev), lambda r: (0, r, 0)),
            pl.BlockSpec((ROWS_BLK, 1), lambda r: (r, 0)),
            pl.BlockSpec((ROWS_BLK, din), lambda r: (r, 0)),
            pl.BlockSpec((din, dout), lambda r: (0, 0)),
            pl.BlockSpec((1, dout), lambda r: (0, 0)),
            pl.BlockSpec((dout, dp_next), lambda r: (0, 0)),
        ],
        out_specs=(
            pl.BlockSpec((ROWS_BLK, dout), lambda r: (r, 0)),
            pl.BlockSpec((ROWS_BLK, dp_next), lambda r: (r, 0)),
        ),
        out_shape=(
            jax.ShapeDtypeStruct((N, dout), jnp.float32),
            jax.ShapeDtypeStruct((N, dp_next), jnp.float32),
        ),
    )(acc, cnt, h, wrt, bl, wlp)


def _tc_fin_body(acc_ref, cnt_ref, h_ref, wr_ref, bl_ref, o_ref):
    s = acc_ref[0] + acc_ref[1]
    o_ref[...] = (s / cnt_ref[...]
                  + jnp.dot(h_ref[...], wr_ref[...],
                            preferred_element_type=jnp.float32)
                  + bl_ref[...])


def _tc_fin(acc, cnt, h, wrt, bl):
    return pl.pallas_call(
        _tc_fin_body,
        grid=(GRID,),
        in_specs=[
            pl.BlockSpec((NC, ROWS_BLK, 16), lambda r: (0, r, 0)),
            pl.BlockSpec((ROWS_BLK, 1), lambda r: (r, 0)),
            pl.BlockSpec((ROWS_BLK, 28), lambda r: (r, 0)),
            pl.BlockSpec((28, 16), lambda r: (0, 0)),
            pl.BlockSpec((1, 16), lambda r: (0, 0)),
        ],
        out_specs=pl.BlockSpec((ROWS_BLK, 16), lambda r: (r, 0)),
        out_shape=jax.ShapeDtypeStruct((N, 16), jnp.float32),
    )(acc, cnt, h, wrt, bl)


def kernel(x, edge_index, Wl0, bl0, Wr0, Wl1, bl1, Wr1, Wl2, bl2, Wr2,
           Wl3, bl3, Wr3):
    ei = edge_index.astype(jnp.int32)
    src = jnp.concatenate([ei[0], jnp.zeros((E_PAD - E,), jnp.int32)])
    dst = jnp.concatenate(
        [ei[1], jnp.full((E_PAD - E,), DUMMY_DST, jnp.int32)])
    src3 = src.reshape(NW, EPW, CHUNK)
    dst3 = dst.reshape(NW, EPW, CHUNK)

    wl0p = jnp.pad(Wl0.T, ((0, 0), (0, 96 - 85)))
    ones_row = jnp.zeros((1, 96), jnp.float32).at[0, 85].set(1.0)
    wl1p = jnp.pad(Wl1.T, ((0, 0), (0, 64 - 56)))
    wl2p = jnp.pad(Wl2.T, ((0, 0), (0, 32 - 28)))
    wl3p = jnp.pad(Wl3.T, ((0, 0), (0, 16 - 1)))
    wr3p = jnp.pad(Wr3.T, ((0, 0), (0, 16 - 1)))
    bl3p = jnp.pad(bl3.reshape(1, -1), ((0, 0), (0, 16 - 1)))

    z96 = jnp.zeros((N_PAD, 96), jnp.float32)
    z64 = jnp.zeros((N_PAD, 64), jnp.float32)
    z32 = jnp.zeros((N_PAD, 32), jnp.float32)
    z16 = jnp.zeros((N_PAD, 16), jnp.float32)

    proj0 = _tc0(x, wl0p, ones_row)
    acc0 = _SC_AGG[96](proj0, src3, dst3, z96)
    h1, p1, cnt = _tc1(acc0, x, Wr0.T, bl0.reshape(1, -1), wl1p)
    acc1 = _SC_AGG[64](p1, src3, dst3, z64)
    h2, p2 = _tc_mid(acc1, cnt, h1, Wr1.T, bl1.reshape(1, -1), wl2p,
                     64, 56, 85, 56, 32)
    acc2 = _SC_AGG[32](p2, src3, dst3, z32)
    h3, p3 = _tc_mid(acc2, cnt, h2, Wr2.T, bl2.reshape(1, -1), wl3p,
                     32, 28, 56, 28, 16)
    acc3 = _SC_AGG[16](p3, src3, dst3, z16)
    out = _tc_fin(acc3, cnt, h3, wr3p, bl3p)
    return out[:, :1]
